# baseline (speedup 1.0000x reference)
import functools

import jax
import jax.numpy as jnp
from jax import lax
from jax.experimental import pallas as pl
from jax.experimental.pallas import tpu as pltpu
from jax.experimental.pallas import tpu_sc as plsc

V = 2048
D = 1024
DFF = 2048
E = 8
NS = 2
DOUT = 16384
T = 4096

VB = 512
JB = 1024
DW = D // 2


def _pack_bf16_pairs(x):
    r = lax.bitcast_convert_type(x, jnp.int32)
    b16 = (r + 0x7FFF + ((r >> 16) & 1)) >> 16
    lo = b16[:, :DW] & 0xFFFF
    hi = b16[:, DW:] << 16
    return lo | hi


def _unpack_bf16_pairs(p):
    lo = lax.bitcast_convert_type(p << 16, jnp.float32)
    hi = lax.bitcast_convert_type(p & jnp.int32(-65536), jnp.float32)
    return lo.astype(jnp.bfloat16), hi.astype(jnp.bfloat16)


def _expert_body(emb_ref, rw_ref, wg_ref, wu_ref, wd_ref, out_ref,
                 wgb_ref, wub_ref, wdb_ref):
    @pl.when(pl.program_id(0) == 0)
    def _cast_weights_once():
        wgb_ref[...] = wg_ref[...].astype(jnp.bfloat16)
        wub_ref[...] = wu_ref[...].astype(jnp.bfloat16)
        wdb_ref[...] = wd_ref[...].astype(jnp.bfloat16)

    x = emb_ref[...]
    logits = lax.dot_general(x, rw_ref[...], (((1,), (1,)), ((), ())),
                             preferred_element_type=jnp.float32)
    iota = lax.broadcasted_iota(jnp.int32, (VB, E), 1)
    m1 = jnp.max(logits, axis=1, keepdims=True)
    i1 = jnp.min(jnp.where(logits == m1, iota, E), axis=1, keepdims=True)
    l2 = jnp.where(iota == i1, -jnp.inf, logits)
    m2 = jnp.max(l2, axis=1, keepdims=True)
    i2 = jnp.min(jnp.where(l2 == m2, iota, E), axis=1, keepdims=True)
    e2 = jnp.exp(m2 - m1)
    a = (jnp.where(i1 < NS, 1.0, 0.0) + e2 * jnp.where(i2 < NS, 1.0, 0.0)) \
        / (1.0 + e2)
    xb = x.astype(jnp.bfloat16)
    gate = lax.dot_general(xb, wgb_ref[...],
                           (((1,), (1,)), ((), ())),
                           preferred_element_type=jnp.float32)
    up = lax.dot_general(xb, wub_ref[...],
                         (((1,), (1,)), ((), ())),
                         preferred_element_type=jnp.float32)
    h = gate * jax.nn.sigmoid(gate) * up
    mlp = lax.dot_general(h.astype(jnp.bfloat16),
                          wdb_ref[...],
                          (((1,), (1,)), ((), ())),
                          preferred_element_type=jnp.float32)
    out_ref[...] = _pack_bf16_pairs(a * x + (1.0 - a) * mlp)


def _combined_table(embedding, router_w, w_gate, w_up, w_down):
    return pl.pallas_call(
        _expert_body,
        grid=(V // VB,),
        in_specs=[
            pl.BlockSpec((VB, D), lambda i: (i, 0)),
            pl.BlockSpec((E, D), lambda i: (0, 0)),
            pl.BlockSpec((DFF, D), lambda i: (0, 0)),
            pl.BlockSpec((DFF, D), lambda i: (0, 0)),
            pl.BlockSpec((D, DFF), lambda i: (0, 0)),
        ],
        out_specs=pl.BlockSpec((VB, DW), lambda i: (i, 0)),
        out_shape=jax.ShapeDtypeStruct((V, DW), jnp.int32),
        scratch_shapes=[
            pltpu.VMEM((DFF, D), jnp.bfloat16),
            pltpu.VMEM((DFF, D), jnp.bfloat16),
            pltpu.VMEM((D, DFF), jnp.bfloat16),
        ],
    )(embedding, router_w, w_gate, w_up, w_down)


def _sc_gather(table, idx):
    info = plsc.get_sparse_core_info()
    nc, ns = info.num_cores, info.num_subcores
    nw = nc * ns
    b_per_w = T // nw
    chunk = b_per_w
    mesh = plsc.VectorSubcoreMesh(core_axis_name="c", subcore_axis_name="s")

    @functools.partial(
        pl.kernel, mesh=mesh,
        out_type=jax.ShapeDtypeStruct((T, DW), jnp.int32),
        scratch_types=[
            pltpu.VMEM((chunk,), jnp.int32),
            pltpu.VMEM((chunk, DW), jnp.int32),
            pltpu.SemaphoreType.DMA,
        ],
    )
    def gather_k(table_hbm, idx_hbm, out_hbm, idx_v, rows_v, sem):
        wid = lax.axis_index("s") * nc + lax.axis_index("c")
        for c in range(b_per_w // chunk):
            base = wid * b_per_w + c * chunk
            pltpu.sync_copy(idx_hbm.at[pl.ds(base, chunk)], idx_v)
            pltpu.async_copy(table_hbm.at[idx_v], rows_v, sem).wait()
            pltpu.sync_copy(rows_v, out_hbm.at[pl.ds(base, chunk)])

    return gather_k(table, idx)


TB = 2048


def _out_body(xg_ref, wt_ref, b_ref, out_ref, xgb_ref):
    j = pl.program_id(0)
    i = pl.program_id(1)

    @pl.when(jnp.logical_and(j == 0, i == 0))
    def _unpack_once():
        lo, hi = _unpack_bf16_pairs(xg_ref[...])
        xgb_ref[:, :DW] = lo
        xgb_ref[:, DW:] = hi

    acc = lax.dot_general(xgb_ref[pl.ds(i * TB, TB), :],
                          wt_ref[...].astype(jnp.bfloat16),
                          (((1,), (1,)), ((), ())),
                          preferred_element_type=jnp.float32)
    out_ref[...] = acc + b_ref[pl.ds(j, 1), :]


def _final_transform(xg, w_t, b_t):
    b2d = b_t.reshape(DOUT // JB, JB)
    return pl.pallas_call(
        _out_body,
        grid=(DOUT // JB, T // TB),
        in_specs=[
            pl.BlockSpec((T, DW), lambda j, i: (0, 0)),
            pl.BlockSpec((JB, D), lambda j, i: (j, 0)),
            pl.BlockSpec((DOUT // JB, JB), lambda j, i: (0, 0)),
        ],
        out_specs=pl.BlockSpec((TB, JB), lambda j, i: (i, j)),
        out_shape=jax.ShapeDtypeStruct((T, DOUT), jnp.float32),
        scratch_shapes=[pltpu.VMEM((T, D), jnp.bfloat16)],
    )(xg, w_t, b2d)


def kernel(indices, embedding, router_w, w_gate, w_up, w_down, w_t, b_t):
    B, S = indices.shape
    idx = indices.reshape(-1).astype(jnp.int32)
    combined = _combined_table(embedding, router_w, w_gate, w_up, w_down)
    xg = _sc_gather(combined, idx)
    out = _final_transform(xg, w_t, b_t)
    return out.reshape(B, S, DOUT)


# speedup vs baseline: 1.0015x; 1.0015x over previous
import functools

import jax
import jax.numpy as jnp
from jax import lax
from jax.experimental import pallas as pl
from jax.experimental.pallas import tpu as pltpu
from jax.experimental.pallas import tpu_sc as plsc

V = 2048
D = 1024
DFF = 2048
E = 8
NS = 2
DOUT = 16384
T = 4096

VB = 512
JB = 1024
DW = D // 2


def _pack_bf16_pairs(x):
    r = lax.bitcast_convert_type(x, jnp.int32)
    b16 = (r + 0x7FFF + ((r >> 16) & 1)) >> 16
    lo = b16[:, :DW] & 0xFFFF
    hi = b16[:, DW:] << 16
    return lo | hi


def _unpack_bf16_pairs(p):
    lo = lax.bitcast_convert_type(p << 16, jnp.float32)
    hi = lax.bitcast_convert_type(p & jnp.int32(-65536), jnp.float32)
    return lo.astype(jnp.bfloat16), hi.astype(jnp.bfloat16)


def _expert_body(emb_ref, rw_ref, wg_ref, wu_ref, wd_ref, out_ref,
                 wgb_ref, wub_ref, wdb_ref):
    @pl.when(pl.program_id(0) == 0)
    def _cast_weights_once():
        wgb_ref[...] = wg_ref[...].astype(jnp.bfloat16)
        wub_ref[...] = wu_ref[...].astype(jnp.bfloat16)
        wdb_ref[...] = wd_ref[...].astype(jnp.bfloat16)

    x = emb_ref[...]
    logits = lax.dot_general(x, rw_ref[...], (((1,), (1,)), ((), ())),
                             preferred_element_type=jnp.float32)
    iota = lax.broadcasted_iota(jnp.int32, (VB, E), 1)
    m1 = jnp.max(logits, axis=1, keepdims=True)
    i1 = jnp.min(jnp.where(logits == m1, iota, E), axis=1, keepdims=True)
    l2 = jnp.where(iota == i1, -jnp.inf, logits)
    m2 = jnp.max(l2, axis=1, keepdims=True)
    i2 = jnp.min(jnp.where(l2 == m2, iota, E), axis=1, keepdims=True)
    e2 = jnp.exp(m2 - m1)
    a = (jnp.where(i1 < NS, 1.0, 0.0) + e2 * jnp.where(i2 < NS, 1.0, 0.0)) \
        / (1.0 + e2)
    xb = x.astype(jnp.bfloat16)
    gate = lax.dot_general(xb, wgb_ref[...],
                           (((1,), (1,)), ((), ())),
                           preferred_element_type=jnp.float32)
    up = lax.dot_general(xb, wub_ref[...],
                         (((1,), (1,)), ((), ())),
                         preferred_element_type=jnp.float32)
    h = gate * jax.nn.sigmoid(gate) * up
    mlp = lax.dot_general(h.astype(jnp.bfloat16),
                          wdb_ref[...],
                          (((1,), (1,)), ((), ())),
                          preferred_element_type=jnp.float32)
    out_ref[...] = _pack_bf16_pairs(a * x + (1.0 - a) * mlp)


def _combined_table(embedding, router_w, w_gate, w_up, w_down):
    return pl.pallas_call(
        _expert_body,
        grid=(V // VB,),
        in_specs=[
            pl.BlockSpec((VB, D), lambda i: (i, 0)),
            pl.BlockSpec((E, D), lambda i: (0, 0)),
            pl.BlockSpec((DFF, D), lambda i: (0, 0)),
            pl.BlockSpec((DFF, D), lambda i: (0, 0)),
            pl.BlockSpec((D, DFF), lambda i: (0, 0)),
        ],
        out_specs=pl.BlockSpec((VB, DW), lambda i: (i, 0)),
        out_shape=jax.ShapeDtypeStruct((V, DW), jnp.int32),
        scratch_shapes=[
            pltpu.VMEM((DFF, D), jnp.bfloat16),
            pltpu.VMEM((DFF, D), jnp.bfloat16),
            pltpu.VMEM((D, DFF), jnp.bfloat16),
        ],
    )(embedding, router_w, w_gate, w_up, w_down)


def _sc_gather(table, idx):
    info = plsc.get_sparse_core_info()
    nc, ns = info.num_cores, info.num_subcores
    nw = nc * ns
    b_per_w = T // nw
    chunk = b_per_w
    mesh = plsc.VectorSubcoreMesh(core_axis_name="c", subcore_axis_name="s")

    @functools.partial(
        pl.kernel, mesh=mesh,
        out_type=jax.ShapeDtypeStruct((T, DW), jnp.int32),
        scratch_types=[
            pltpu.VMEM((chunk,), jnp.int32),
            pltpu.VMEM((chunk, DW), jnp.int32),
            pltpu.SemaphoreType.DMA,
        ],
    )
    def gather_k(table_hbm, idx_hbm, out_hbm, idx_v, rows_v, sem):
        wid = lax.axis_index("s") * nc + lax.axis_index("c")
        for c in range(b_per_w // chunk):
            base = wid * b_per_w + c * chunk
            pltpu.sync_copy(idx_hbm.at[pl.ds(base, chunk)], idx_v)
            pltpu.async_copy(table_hbm.at[idx_v], rows_v, sem).wait()
            pltpu.sync_copy(rows_v, out_hbm.at[pl.ds(base, chunk)])

    return gather_k(table, idx)


TB = 2048
NBUF = 2


def _out_body(xg_ref, wt_ref, b_ref, out_hbm, xgb_ref, acc_ref, sem):
    j = pl.program_id(0)
    i = pl.program_id(1)
    ni = pl.num_programs(1)
    s = j * ni + i
    slot = lax.rem(s, NBUF)

    @pl.when(s == 0)
    def _unpack_once():
        lo, hi = _unpack_bf16_pairs(xg_ref[...])
        xgb_ref[:, :DW] = lo
        xgb_ref[:, DW:] = hi

    dst = out_hbm.at[pl.ds(i * TB, TB), pl.ds(j * JB, JB)]

    @pl.when(s >= NBUF)
    def _drain_old():
        pltpu.make_async_copy(acc_ref.at[slot], dst, sem.at[slot]).wait()

    acc_ref[slot] = lax.dot_general(
        xgb_ref[pl.ds(i * TB, TB), :],
        wt_ref[...].astype(jnp.bfloat16),
        (((1,), (1,)), ((), ())),
        preferred_element_type=jnp.float32) + b_ref[pl.ds(j, 1), :]
    pltpu.make_async_copy(acc_ref.at[slot], dst, sem.at[slot]).start()

    nj = pl.num_programs(0)

    @pl.when(s == nj * ni - 1)
    def _drain_all():
        for k in range(NBUF):
            pltpu.make_async_copy(acc_ref.at[k], dst, sem.at[k]).wait()


def _final_transform(xg, w_t, b_t):
    b2d = b_t.reshape(DOUT // JB, JB)
    return pl.pallas_call(
        _out_body,
        grid=(DOUT // JB, T // TB),
        in_specs=[
            pl.BlockSpec((T, DW), lambda j, i: (0, 0)),
            pl.BlockSpec((JB, D), lambda j, i: (j, 0)),
            pl.BlockSpec((DOUT // JB, JB), lambda j, i: (0, 0)),
        ],
        out_specs=pl.BlockSpec(memory_space=pl.ANY),
        out_shape=jax.ShapeDtypeStruct((T, DOUT), jnp.float32),
        scratch_shapes=[
            pltpu.VMEM((T, D), jnp.bfloat16),
            pltpu.VMEM((NBUF, TB, JB), jnp.float32),
            pltpu.SemaphoreType.DMA((NBUF,)),
        ],
    )(xg, w_t, b2d)


def kernel(indices, embedding, router_w, w_gate, w_up, w_down, w_t, b_t):
    B, S = indices.shape
    idx = indices.reshape(-1).astype(jnp.int32)
    combined = _combined_table(embedding, router_w, w_gate, w_up, w_down)
    xg = _sc_gather(combined, idx)
    out = _final_transform(xg, w_t, b_t)
    return out.reshape(B, S, DOUT)


# speedup vs baseline: 1.0054x; 1.0039x over previous
import functools

import jax
import jax.numpy as jnp
from jax import lax
from jax.experimental import pallas as pl
from jax.experimental.pallas import tpu as pltpu
from jax.experimental.pallas import tpu_sc as plsc

V = 2048
D = 1024
DFF = 2048
E = 8
NS = 2
DOUT = 16384
T = 4096

VB = 512
JB = 1024
DW = D // 2


def _pack_bf16_pairs(x):
    r = lax.bitcast_convert_type(x, jnp.int32)
    b16 = (r + 0x7FFF + ((r >> 16) & 1)) >> 16
    lo = b16[:, :DW] & 0xFFFF
    hi = b16[:, DW:] << 16
    return lo | hi


def _unpack_bf16_pairs(p):
    lo = lax.bitcast_convert_type(p << 16, jnp.float32)
    hi = lax.bitcast_convert_type(p & jnp.int32(-65536), jnp.float32)
    return lo.astype(jnp.bfloat16), hi.astype(jnp.bfloat16)


def _expert_body(emb_ref, rw_ref, wg_ref, wu_ref, wd_ref, out_ref):
    x = emb_ref[...]
    logits = lax.dot_general(x, rw_ref[...], (((1,), (1,)), ((), ())),
                             preferred_element_type=jnp.float32)
    iota = lax.broadcasted_iota(jnp.int32, (VB, E), 1)
    m1 = jnp.max(logits, axis=1, keepdims=True)
    i1 = jnp.min(jnp.where(logits == m1, iota, E), axis=1, keepdims=True)
    l2 = jnp.where(iota == i1, -jnp.inf, logits)
    m2 = jnp.max(l2, axis=1, keepdims=True)
    i2 = jnp.min(jnp.where(l2 == m2, iota, E), axis=1, keepdims=True)
    e2 = jnp.exp(m2 - m1)
    a = (jnp.where(i1 < NS, 1.0, 0.0) + e2 * jnp.where(i2 < NS, 1.0, 0.0)) \
        / (1.0 + e2)
    xb = x.astype(jnp.bfloat16)
    gate = lax.dot_general(xb, wg_ref[...].astype(jnp.bfloat16),
                           (((1,), (1,)), ((), ())),
                           preferred_element_type=jnp.float32)
    up = lax.dot_general(xb, wu_ref[...].astype(jnp.bfloat16),
                         (((1,), (1,)), ((), ())),
                         preferred_element_type=jnp.float32)
    h = gate * jax.nn.sigmoid(gate) * up
    mlp = lax.dot_general(h.astype(jnp.bfloat16),
                          wd_ref[...].astype(jnp.bfloat16),
                          (((1,), (1,)), ((), ())),
                          preferred_element_type=jnp.float32)
    out_ref[...] = _pack_bf16_pairs(a * x + (1.0 - a) * mlp)


def _combined_table(embedding, router_w, w_gate, w_up, w_down):
    return pl.pallas_call(
        _expert_body,
        grid=(V // VB,),
        in_specs=[
            pl.BlockSpec((VB, D), lambda i: (i, 0)),
            pl.BlockSpec((E, D), lambda i: (0, 0)),
            pl.BlockSpec((DFF, D), lambda i: (0, 0)),
            pl.BlockSpec((DFF, D), lambda i: (0, 0)),
            pl.BlockSpec((D, DFF), lambda i: (0, 0)),
        ],
        out_specs=pl.BlockSpec((VB, DW), lambda i: (i, 0)),
        out_shape=jax.ShapeDtypeStruct((V, DW), jnp.int32),
    )(embedding, router_w, w_gate, w_up, w_down)


def _sc_gather(table, idx):
    info = plsc.get_sparse_core_info()
    nc, ns = info.num_cores, info.num_subcores
    nw = nc * ns
    b_per_w = T // nw
    chunk = b_per_w
    mesh = plsc.VectorSubcoreMesh(core_axis_name="c", subcore_axis_name="s")

    @functools.partial(
        pl.kernel, mesh=mesh,
        out_type=jax.ShapeDtypeStruct((T, DW), jnp.int32),
        scratch_types=[
            pltpu.VMEM((chunk,), jnp.int32),
            pltpu.VMEM((chunk, DW), jnp.int32),
            pltpu.SemaphoreType.DMA,
        ],
    )
    def gather_k(table_hbm, idx_hbm, out_hbm, idx_v, rows_v, sem):
        wid = lax.axis_index("s") * nc + lax.axis_index("c")
        for c in range(b_per_w // chunk):
            base = wid * b_per_w + c * chunk
            pltpu.sync_copy(idx_hbm.at[pl.ds(base, chunk)], idx_v)
            pltpu.async_copy(table_hbm.at[idx_v], rows_v, sem).wait()
            pltpu.sync_copy(rows_v, out_hbm.at[pl.ds(base, chunk)])

    return gather_k(table, idx)


TB = 2048


def _out_body(xg_ref, wt_ref, b_ref, out_ref, xgb_ref):
    j = pl.program_id(0)
    i = pl.program_id(1)

    @pl.when(jnp.logical_and(j == 0, i == 0))
    def _unpack_once():
        lo, hi = _unpack_bf16_pairs(xg_ref[...])
        xgb_ref[:, :DW] = lo
        xgb_ref[:, DW:] = hi

    acc = lax.dot_general(xgb_ref[pl.ds(i * TB, TB), :],
                          wt_ref[...].astype(jnp.bfloat16),
                          (((1,), (1,)), ((), ())),
                          preferred_element_type=jnp.float32)
    out_ref[...] = acc + b_ref[pl.ds(j, 1), :]


def _final_transform(xg, w_t, b_t):
    b2d = b_t.reshape(DOUT // JB, JB)
    return pl.pallas_call(
        _out_body,
        grid=(DOUT // JB, T // TB),
        in_specs=[
            pl.BlockSpec((T, DW), lambda j, i: (0, 0)),
            pl.BlockSpec((JB, D), lambda j, i: (j, 0)),
            pl.BlockSpec((DOUT // JB, JB), lambda j, i: (0, 0)),
        ],
        out_specs=pl.BlockSpec((TB, JB), lambda j, i: (i, j)),
        out_shape=jax.ShapeDtypeStruct((T, DOUT), jnp.float32),
        scratch_shapes=[pltpu.VMEM((T, D), jnp.bfloat16)],
    )(xg, w_t, b2d)


def kernel(indices, embedding, router_w, w_gate, w_up, w_down, w_t, b_t):
    B, S = indices.shape
    idx = indices.reshape(-1).astype(jnp.int32)
    combined = _combined_table(embedding, router_w, w_gate, w_up, w_down)
    xg = _sc_gather(combined, idx)
    out = _final_transform(xg, w_t, b_t)
    return out.reshape(B, S, DOUT)


# speedup vs baseline: 1.0065x; 1.0011x over previous
import functools

import jax
import jax.numpy as jnp
from jax import lax
from jax.experimental import pallas as pl
from jax.experimental.pallas import tpu as pltpu
from jax.experimental.pallas import tpu_sc as plsc

V = 2048
D = 1024
DFF = 2048
E = 8
NS = 2
DOUT = 16384
T = 4096

VB = 512
JB = 1024
DW = D // 2


def _pack_bf16_pairs(x):
    r = lax.bitcast_convert_type(x, jnp.int32)
    b16 = (r + 0x7FFF + ((r >> 16) & 1)) >> 16
    lo = b16[:, :DW] & 0xFFFF
    hi = b16[:, DW:] << 16
    return lo | hi


def _unpack_bf16_pairs(p):
    lo = lax.bitcast_convert_type(p << 16, jnp.float32)
    hi = lax.bitcast_convert_type(p & jnp.int32(-65536), jnp.float32)
    return lo.astype(jnp.bfloat16), hi.astype(jnp.bfloat16)


def _expert_body(emb_ref, rw_ref, wg_ref, wu_ref, wd_ref, out_ref):
    x = emb_ref[...]
    logits = lax.dot_general(x, rw_ref[...], (((1,), (1,)), ((), ())),
                             preferred_element_type=jnp.float32)
    iota = lax.broadcasted_iota(jnp.int32, (VB, E), 1)
    m1 = jnp.max(logits, axis=1, keepdims=True)
    i1 = jnp.min(jnp.where(logits == m1, iota, E), axis=1, keepdims=True)
    l2 = jnp.where(iota == i1, -jnp.inf, logits)
    m2 = jnp.max(l2, axis=1, keepdims=True)
    i2 = jnp.min(jnp.where(l2 == m2, iota, E), axis=1, keepdims=True)
    e2 = jnp.exp(m2 - m1)
    a = (jnp.where(i1 < NS, 1.0, 0.0) + e2 * jnp.where(i2 < NS, 1.0, 0.0)) \
        / (1.0 + e2)
    xb = x.astype(jnp.bfloat16)
    gate = lax.dot_general(xb, wg_ref[...].astype(jnp.bfloat16),
                           (((1,), (1,)), ((), ())),
                           preferred_element_type=jnp.float32)
    up = lax.dot_general(xb, wu_ref[...].astype(jnp.bfloat16),
                         (((1,), (1,)), ((), ())),
                         preferred_element_type=jnp.float32)
    h = gate * jax.nn.sigmoid(gate) * up
    mlp = lax.dot_general(h.astype(jnp.bfloat16),
                          wd_ref[...].astype(jnp.bfloat16),
                          (((1,), (1,)), ((), ())),
                          preferred_element_type=jnp.float32)
    out_ref[...] = _pack_bf16_pairs(a * x + (1.0 - a) * mlp)


def _combined_table(embedding, router_w, w_gate, w_up, w_down):
    return pl.pallas_call(
        _expert_body,
        grid=(V // VB,),
        in_specs=[
            pl.BlockSpec((VB, D), lambda i: (i, 0)),
            pl.BlockSpec((E, D), lambda i: (0, 0)),
            pl.BlockSpec((DFF, D), lambda i: (0, 0)),
            pl.BlockSpec((DFF, D), lambda i: (0, 0)),
            pl.BlockSpec((D, DFF), lambda i: (0, 0)),
        ],
        out_specs=pl.BlockSpec((VB, DW), lambda i: (i, 0)),
        out_shape=jax.ShapeDtypeStruct((V, DW), jnp.int32),
    )(embedding, router_w, w_gate, w_up, w_down)


def _sc_gather(table, idx):
    info = plsc.get_sparse_core_info()
    nc, ns = info.num_cores, info.num_subcores
    nw = nc * ns
    b_per_w = T // nw
    chunk = b_per_w
    mesh = plsc.VectorSubcoreMesh(core_axis_name="c", subcore_axis_name="s")

    @functools.partial(
        pl.kernel, mesh=mesh,
        out_type=jax.ShapeDtypeStruct((T, DW), jnp.int32),
        scratch_types=[
            pltpu.VMEM((chunk,), jnp.int32),
            pltpu.VMEM((chunk, DW), jnp.int32),
            pltpu.SemaphoreType.DMA,
        ],
    )
    def gather_k(table_hbm, idx_hbm, out_hbm, idx_v, rows_v, sem):
        wid = lax.axis_index("s") * nc + lax.axis_index("c")
        for c in range(b_per_w // chunk):
            base = wid * b_per_w + c * chunk
            pltpu.sync_copy(idx_hbm.at[pl.ds(base, chunk)], idx_v)
            pltpu.async_copy(table_hbm.at[idx_v], rows_v, sem).wait()
            pltpu.sync_copy(rows_v, out_hbm.at[pl.ds(base, chunk)])

    return gather_k(table, idx)


TB = 2048


def _out_body(xg_ref, wt_ref, b_ref, out_ref, xgb_ref):
    j = pl.program_id(0)
    i = pl.program_id(1)

    @pl.when(j == 0)
    def _unpack_own_rows():
        lo, hi = _unpack_bf16_pairs(xg_ref[pl.ds(i * TB, TB), :])
        xgb_ref[pl.ds(i * TB, TB), :DW] = lo
        xgb_ref[pl.ds(i * TB, TB), DW:] = hi

    acc = lax.dot_general(xgb_ref[pl.ds(i * TB, TB), :],
                          wt_ref[...].astype(jnp.bfloat16),
                          (((1,), (1,)), ((), ())),
                          preferred_element_type=jnp.float32)
    out_ref[...] = acc + b_ref[pl.ds(j, 1), :]


def _final_transform(xg, w_t, b_t):
    b2d = b_t.reshape(DOUT // JB, JB)
    return pl.pallas_call(
        _out_body,
        grid=(DOUT // JB, T // TB),
        in_specs=[
            pl.BlockSpec((T, DW), lambda j, i: (0, 0)),
            pl.BlockSpec((JB, D), lambda j, i: (j, 0)),
            pl.BlockSpec((DOUT // JB, JB), lambda j, i: (0, 0)),
        ],
        out_specs=pl.BlockSpec((TB, JB), lambda j, i: (i, j)),
        out_shape=jax.ShapeDtypeStruct((T, DOUT), jnp.float32),
        scratch_shapes=[pltpu.VMEM((T, D), jnp.bfloat16)],
    )(xg, w_t, b2d)


def kernel(indices, embedding, router_w, w_gate, w_up, w_down, w_t, b_t):
    B, S = indices.shape
    idx = indices.reshape(-1).astype(jnp.int32)
    combined = _combined_table(embedding, router_w, w_gate, w_up, w_down)
    xg = _sc_gather(combined, idx)
    out = _final_transform(xg, w_t, b_t)
    return out.reshape(B, S, DOUT)
